# Initial kernel scaffold; baseline (speedup 1.0000x reference)
import jax
import jax.numpy as jnp
from jax.experimental import pallas as pl


def kernel(x, edge_index, edge_attr, W1, b1, W2, b2, We1, be1, We2, be2):
    raise NotImplementedError("write your pallas kernel here")


# speedup vs baseline: 2.2620x; 2.2620x over previous
import functools

import jax
import jax.numpy as jnp
from jax import lax
from jax.experimental import pallas as pl
from jax.experimental.pallas import tpu as pltpu
from jax.experimental.pallas import tpu_sc as plsc

F32 = jnp.float32


def _prep_body(x_ref, wea_ref, web_ref, be1_ref, w1a_ref, b1_ref,
               pa_ref, pb_ref, q_ref):
    xb = x_ref[...]
    pa_ref[...] = jnp.dot(xb, wea_ref[...], preferred_element_type=F32) + be1_ref[...]
    pb_ref[...] = jnp.dot(xb, web_ref[...], preferred_element_type=F32)
    q_ref[...] = jnp.dot(xb, w1a_ref[...], preferred_element_type=F32) + b1_ref[...]


def _edge_body(apa_ref, apb_ref, ea_ref, qr_ref, we1c_ref, we2_ref, be2_ref,
               w1b_ref, ean_ref, hn_ref):
    t = apa_ref[...] + apb_ref[...] + jnp.dot(
        ea_ref[...], we1c_ref[...], preferred_element_type=F32)
    he = jax.nn.softplus(t)
    ean = jnp.dot(he, we2_ref[...], preferred_element_type=F32) + be2_ref[...]
    ean_ref[...] = ean
    u = qr_ref[...] + jnp.dot(ean, w1b_ref[...], preferred_element_type=F32)
    hn_ref[...] = jax.nn.softplus(u).astype(jnp.bfloat16)


def _final_body(s_ref, c_ref, w2_ref, b2_ref, out_ref):
    sacc = s_ref[0].astype(F32) + s_ref[1].astype(F32)
    cnt = c_ref[0, :, 0:1] + c_ref[1, :, 0:1]
    out_ref[...] = (jnp.dot(sacc, w2_ref[...], preferred_element_type=F32)
                    + cnt * b2_ref[...])


def _make_gather(n, e, d, ed):
    g = 128
    ng = e // g
    nt = 32
    jmax = (ng + nt - 1) // nt
    mesh = plsc.VectorSubcoreMesh(core_axis_name="c", subcore_axis_name="s",
                                  num_cores=2, num_subcores=16)

    @functools.partial(
        pl.kernel,
        out_type=[jax.ShapeDtypeStruct((e, ed), F32),
                  jax.ShapeDtypeStruct((e, ed), F32),
                  jax.ShapeDtypeStruct((e, d), F32)],
        mesh=mesh,
        compiler_params=pltpu.CompilerParams(use_tc_tiling_on_sc=False),
        scratch_types=[pltpu.VMEM((g,), jnp.int32),
                       pltpu.VMEM((g,), jnp.int32),
                       pltpu.VMEM((g, ed), F32),
                       pltpu.VMEM((g, ed), F32),
                       pltpu.VMEM((g, d), F32),
                       pltpu.SemaphoreType.DMA,
                       pltpu.SemaphoreType.DMA,
                       pltpu.SemaphoreType.DMA],
    )
    def gather(pa_hbm, pb_hbm, q_hbm, row_hbm, col_hbm,
               apa_hbm, apb_hbm, qr_hbm, ir, ic, bpa, bpb, bq, s1, s2, s3):
        wid = lax.axis_index("s") * 2 + lax.axis_index("c")

        @pl.loop(0, jmax)
        def _(j):
            grp = wid + nt * j

            @pl.when(grp < ng)
            def _():
                base = grp * g
                pltpu.sync_copy(row_hbm.at[pl.ds(base, g)], ir)
                pltpu.sync_copy(col_hbm.at[pl.ds(base, g)], ic)
                ca = pltpu.async_copy(pa_hbm.at[ir], bpa, s1)
                cb = pltpu.async_copy(pb_hbm.at[ic], bpb, s2)
                cq = pltpu.async_copy(q_hbm.at[ir], bq, s3)
                ca.wait()
                cb.wait()
                cq.wait()
                pltpu.sync_copy(bpa, apa_hbm.at[pl.ds(base, g)])
                pltpu.sync_copy(bpb, apb_hbm.at[pl.ds(base, g)])
                pltpu.sync_copy(bq, qr_hbm.at[pl.ds(base, g)])

    return gather


def _make_scatter(n, e, d, ed):
    g = 128
    ng = e // g
    nt = 32
    jmax = (ng + nt - 1) // nt
    rpt = n // 16
    rb = rpt // 5
    mesh = plsc.VectorSubcoreMesh(core_axis_name="c", subcore_axis_name="s",
                                  num_cores=2, num_subcores=16)

    @functools.partial(
        pl.kernel,
        out_type=[jax.ShapeDtypeStruct((2 * n, d), jnp.bfloat16),
                  jax.ShapeDtypeStruct((2 * n, ed), F32)],
        mesh=mesh,
        compiler_params=pltpu.CompilerParams(use_tc_tiling_on_sc=False),
        scratch_types=[pltpu.VMEM((g,), jnp.int32),
                       pltpu.VMEM((g, d), jnp.bfloat16),
                       pltpu.VMEM((g, ed), F32),
                       pltpu.VMEM((rb, d), jnp.bfloat16),
                       pltpu.VMEM((rpt, ed), F32),
                       pltpu.VMEM_SHARED((n, d), jnp.bfloat16),
                       pltpu.VMEM_SHARED((n, ed), F32)],
    )
    def scatter(hn_hbm, col_hbm, s2_hbm, c2_hbm,
                ic, bh, ones, zb, cz, s_sh, c_sh):
        cid = lax.axis_index("c")
        sid = lax.axis_index("s")
        wid = sid * 2 + cid

        zvec = jnp.zeros((16,), F32)
        zvec16 = jnp.zeros((32,), jnp.bfloat16)
        onev = jnp.ones((16,), F32)

        @pl.loop(0, rb)
        def _(i):
            for k in range(d // 32):
                zb[i, pl.ds(k * 32, 32)] = zvec16

        @pl.loop(0, rpt)
        def _(i):
            cz[i, :] = zvec

        @pl.loop(0, g)
        def _(i):
            ones[i, :] = onev

        r0 = sid * rpt

        @pl.loop(0, 5)
        def _(k):
            pltpu.sync_copy(zb, s_sh.at[pl.ds(r0 + k * rb, rb)])

        pltpu.sync_copy(cz, c_sh.at[pl.ds(r0, rpt)])
        plsc.subcore_barrier()

        @pl.loop(0, jmax)
        def _(j):
            grp = wid + nt * j

            @pl.when(grp < ng)
            def _():
                base = grp * g
                pltpu.sync_copy(col_hbm.at[pl.ds(base, g)], ic)
                pltpu.sync_copy(hn_hbm.at[pl.ds(base, g)], bh)
                pltpu.sync_copy(bh, s_sh.at[ic], add=True)
                pltpu.sync_copy(ones, c_sh.at[ic], add=True)

        plsc.subcore_barrier()

        @pl.loop(0, 5)
        def _(k):
            r = r0 + k * rb
            pltpu.sync_copy(s_sh.at[pl.ds(r, rb)], zb)
            pltpu.sync_copy(zb, s2_hbm.at[pl.ds(cid * n + r, rb)])

        pltpu.sync_copy(c_sh.at[pl.ds(r0, rpt)], cz)
        pltpu.sync_copy(cz, c2_hbm.at[pl.ds(cid * n + r0, rpt)])

    return scatter


def kernel(x, edge_index, edge_attr, W1, b1, W2, b2, We1, be1, We2, be2):
    n, d = x.shape
    e, ed = edge_attr.shape
    row = edge_index[0]
    col = edge_index[1]

    we1a = We1[:d]
    we1b = We1[d:2 * d]
    we1c = We1[2 * d:]
    w1a = W1[:d]
    w1b = W1[d:]

    nb = 5
    bn = n // nb
    pa, pb, q = pl.pallas_call(
        _prep_body,
        grid=(nb,),
        in_specs=[pl.BlockSpec((bn, d), lambda i: (i, 0)),
                  pl.BlockSpec((d, ed), lambda i: (0, 0)),
                  pl.BlockSpec((d, ed), lambda i: (0, 0)),
                  pl.BlockSpec((1, ed), lambda i: (0, 0)),
                  pl.BlockSpec((d, d), lambda i: (0, 0)),
                  pl.BlockSpec((1, d), lambda i: (0, 0))],
        out_specs=[pl.BlockSpec((bn, ed), lambda i: (i, 0)),
                   pl.BlockSpec((bn, ed), lambda i: (i, 0)),
                   pl.BlockSpec((bn, d), lambda i: (i, 0))],
        out_shape=[jax.ShapeDtypeStruct((n, ed), F32),
                   jax.ShapeDtypeStruct((n, ed), F32),
                   jax.ShapeDtypeStruct((n, d), F32)],
    )(x, we1a, we1b, be1.reshape(1, ed), w1a, b1.reshape(1, d))

    apa, apb, qr = _make_gather(n, e, d, ed)(pa, pb, q, row, col)

    eb = 2560
    neb = e // eb
    ean, hn = pl.pallas_call(
        _edge_body,
        grid=(neb,),
        in_specs=[pl.BlockSpec((eb, ed), lambda i: (i, 0)),
                  pl.BlockSpec((eb, ed), lambda i: (i, 0)),
                  pl.BlockSpec((eb, ed), lambda i: (i, 0)),
                  pl.BlockSpec((eb, d), lambda i: (i, 0)),
                  pl.BlockSpec((ed, ed), lambda i: (0, 0)),
                  pl.BlockSpec((ed, ed), lambda i: (0, 0)),
                  pl.BlockSpec((1, ed), lambda i: (0, 0)),
                  pl.BlockSpec((ed, d), lambda i: (0, 0))],
        out_specs=[pl.BlockSpec((eb, ed), lambda i: (i, 0)),
                   pl.BlockSpec((eb, d), lambda i: (i, 0))],
        out_shape=[jax.ShapeDtypeStruct((e, ed), F32),
                   jax.ShapeDtypeStruct((e, d), jnp.bfloat16)],
    )(apa, apb, edge_attr, qr, we1c, We2, be2.reshape(1, ed), w1b)

    s2, c2 = _make_scatter(n, e, d, ed)(hn, col)
    s2 = s2.reshape(2, n, d)
    c2 = c2.reshape(2, n, ed)

    x_new = pl.pallas_call(
        _final_body,
        grid=(nb,),
        in_specs=[pl.BlockSpec((2, bn, d), lambda i: (0, i, 0)),
                  pl.BlockSpec((2, bn, ed), lambda i: (0, i, 0)),
                  pl.BlockSpec((d, d), lambda i: (0, 0)),
                  pl.BlockSpec((1, d), lambda i: (0, 0))],
        out_specs=pl.BlockSpec((bn, d), lambda i: (i, 0)),
        out_shape=jax.ShapeDtypeStruct((n, d), F32),
    )(s2, c2, W2, b2.reshape(1, d))

    return (x_new, ean)
